# baseline (speedup 1.0000x reference)
import functools

import jax
import jax.numpy as jnp
from jax import lax
from jax.experimental import pallas as pl
from jax.experimental.pallas import tpu as pltpu
from jax.experimental.pallas import tpu_sc as plsc

_LANES = 16


def _sc_gather_call(n_elems, n_table, num_workers, num_cores):
    per_w = n_elems // num_workers
    mesh = plsc.VectorSubcoreMesh(core_axis_name="c", subcore_axis_name="s")

    @functools.partial(
        pl.kernel,
        mesh=mesh,
        out_type=jax.ShapeDtypeStruct((n_elems,), jnp.float32),
        scratch_types=[
            pltpu.VMEM((n_table,), jnp.float32),
            pltpu.VMEM((per_w,), jnp.int32),
            pltpu.VMEM((per_w,), jnp.float32),
        ],
        compiler_params=pltpu.CompilerParams(needs_layout_passes=False),
    )
    def sc_gather(table_hbm, idx_hbm, out_hbm, table_v, idx_v, out_v):
        wid = lax.axis_index("s") * num_cores + lax.axis_index("c")
        base = wid * per_w
        pltpu.sync_copy(table_hbm, table_v)
        pltpu.sync_copy(idx_hbm.at[pl.ds(base, per_w)], idx_v)

        def body(i, carry):
            off = pl.multiple_of(i * _LANES, _LANES)
            iv = idx_v[pl.ds(off, _LANES)]
            out_v[pl.ds(off, _LANES)] = plsc.load_gather(table_v, [iv])
            return carry

        lax.fori_loop(0, per_w // _LANES, body, 0)
        pltpu.sync_copy(out_v, out_hbm.at[pl.ds(base, per_w)])

    return sc_gather


def kernel(propensities, position):
    b, l = position.shape
    n = b * l
    idx = position.reshape(n).astype(jnp.int32)
    table = propensities.astype(jnp.float32)

    info = plsc.get_sparse_core_info()
    num_workers = info.num_cores * info.num_subcores
    fn = _sc_gather_call(n, table.shape[0], num_workers, info.num_cores)
    out = fn(table, idx)
    return out.reshape(b, l)


# speedup vs baseline: 1.1225x; 1.1225x over previous
import functools

import jax
import jax.numpy as jnp
from jax import lax
from jax.experimental import pallas as pl
from jax.experimental.pallas import tpu as pltpu
from jax.experimental.pallas import tpu_sc as plsc

_LANES = 16


def _sc_gather_call(n_elems, n_table, num_workers, num_cores):
    per_w = n_elems // num_workers
    mesh = plsc.VectorSubcoreMesh(core_axis_name="c", subcore_axis_name="s")

    @functools.partial(
        pl.kernel,
        mesh=mesh,
        out_type=jax.ShapeDtypeStruct((n_elems,), jnp.float32),
        scratch_types=[
            pltpu.VMEM((n_table,), jnp.float32),
            pltpu.VMEM((per_w,), jnp.int32),
            pltpu.VMEM((per_w,), jnp.float32),
        ],
        compiler_params=pltpu.CompilerParams(needs_layout_passes=False),
    )
    def sc_gather(table_hbm, idx_hbm, out_hbm, table_v, idx_v, out_v):
        wid = lax.axis_index("s") * num_cores + lax.axis_index("c")
        base = wid * per_w
        pltpu.sync_copy(table_hbm, table_v)
        pltpu.sync_copy(idx_hbm.at[pl.ds(base, per_w)], idx_v)

        @plsc.parallel_loop(0, per_w, step=_LANES, unroll=8)
        def body(off):
            iv = idx_v[pl.ds(off, _LANES)]
            out_v[pl.ds(off, _LANES)] = plsc.load_gather(table_v, [iv])
        pltpu.sync_copy(out_v, out_hbm.at[pl.ds(base, per_w)])

    return sc_gather


def kernel(propensities, position):
    b, l = position.shape
    n = b * l
    idx = position.reshape(n).astype(jnp.int32)
    table = propensities.astype(jnp.float32)

    info = plsc.get_sparse_core_info()
    num_workers = info.num_cores * info.num_subcores
    fn = _sc_gather_call(n, table.shape[0], num_workers, info.num_cores)
    out = fn(table, idx)
    return out.reshape(b, l)


# speedup vs baseline: 1.6698x; 1.4876x over previous
import functools

import jax
import jax.numpy as jnp
from jax import lax
from jax.experimental import pallas as pl
from jax.experimental.pallas import tpu as pltpu
from jax.experimental.pallas import tpu_sc as plsc

_LANES = 16
_CHUNK = 256


def _sc_gather_call(n_rows, n_cols, n_table, num_workers, num_cores):
    rows_per_w = n_rows // num_workers
    n_chunks = rows_per_w // _CHUNK
    col_offs = []
    c = 0
    while c + _LANES <= n_cols:
        col_offs.append(c)
        c += _LANES
    if c < n_cols:
        col_offs.append(n_cols - _LANES)
    mesh = plsc.VectorSubcoreMesh(core_axis_name="c", subcore_axis_name="s")

    @functools.partial(
        pl.kernel,
        mesh=mesh,
        out_type=jax.ShapeDtypeStruct((n_rows, n_cols), jnp.float32),
        scratch_types=[
            pltpu.VMEM((n_table,), jnp.float32),
            pltpu.VMEM((_CHUNK, n_cols), jnp.int32),
            pltpu.VMEM((_CHUNK, n_cols), jnp.float32),
        ],
        compiler_params=pltpu.CompilerParams(needs_layout_passes=False),
    )
    def sc_gather(table_hbm, idx_hbm, out_hbm, table_v, idx_v, out_v):
        wid = lax.axis_index("s") * num_cores + lax.axis_index("c")
        base = wid * rows_per_w
        pltpu.sync_copy(table_hbm, table_v)

        def chunk_body(k, carry):
            cbase = base + k * _CHUNK
            pltpu.sync_copy(idx_hbm.at[pl.ds(cbase, _CHUNK)], idx_v)

            @plsc.parallel_loop(0, _CHUNK, step=1, unroll=4)
            def body(r):
                for c in col_offs:
                    iv = idx_v[r, pl.ds(c, _LANES)]
                    out_v[r, pl.ds(c, _LANES)] = plsc.load_gather(table_v, [iv])

            pltpu.sync_copy(out_v, out_hbm.at[pl.ds(cbase, _CHUNK)])
            return carry

        lax.fori_loop(0, n_chunks, chunk_body, 0)

    return sc_gather


def kernel(propensities, position):
    n_rows, n_cols = position.shape
    idx = position.astype(jnp.int32)
    table = propensities.astype(jnp.float32)

    info = plsc.get_sparse_core_info()
    num_workers = info.num_cores * info.num_subcores
    fn = _sc_gather_call(n_rows, n_cols, table.shape[0], num_workers, info.num_cores)
    return fn(table, idx)


# speedup vs baseline: 1.7252x; 1.0332x over previous
import functools

import jax
import jax.numpy as jnp
from jax import lax
from jax.experimental import pallas as pl
from jax.experimental.pallas import tpu as pltpu
from jax.experimental.pallas import tpu_sc as plsc

_LANES = 16
_CHUNK = 128


def _sc_gather_call(n_rows, n_cols, n_table, num_workers, num_cores):
    rows_per_w = n_rows // num_workers
    n_chunks = rows_per_w // _CHUNK
    col_offs = []
    c = 0
    while c + _LANES <= n_cols:
        col_offs.append(c)
        c += _LANES
    if c < n_cols:
        col_offs.append(n_cols - _LANES)
    mesh = plsc.VectorSubcoreMesh(core_axis_name="c", subcore_axis_name="s")

    @functools.partial(
        pl.kernel,
        mesh=mesh,
        out_type=jax.ShapeDtypeStruct((n_rows, n_cols), jnp.float32),
        scratch_types=[
            pltpu.VMEM((n_table,), jnp.float32),
            pltpu.VMEM((_CHUNK, n_cols), jnp.int32),
            pltpu.VMEM((_CHUNK, n_cols), jnp.int32),
            pltpu.VMEM((_CHUNK, n_cols), jnp.float32),
            pltpu.VMEM((_CHUNK, n_cols), jnp.float32),
            pltpu.SemaphoreType.DMA,
            pltpu.SemaphoreType.DMA,
            pltpu.SemaphoreType.DMA,
            pltpu.SemaphoreType.DMA,
        ],
        compiler_params=pltpu.CompilerParams(
            needs_layout_passes=False, disable_bounds_checks=True
        ),
    )
    def sc_gather(
        table_hbm, idx_hbm, out_hbm,
        table_v, idx_v0, idx_v1, out_v0, out_v1, sin0, sin1, sout0, sout1,
    ):
        idx_bufs = (idx_v0, idx_v1)
        out_bufs = (out_v0, out_v1)
        sins = (sin0, sin1)
        souts = (sout0, sout1)
        wid = lax.axis_index("s") * num_cores + lax.axis_index("c")
        base = wid * rows_per_w
        pltpu.sync_copy(table_hbm, table_v)

        in_cps = [None, None]
        out_cps = [None, None]
        in_cps[0] = pltpu.async_copy(
            idx_hbm.at[pl.ds(base, _CHUNK)], idx_bufs[0], sins[0]
        )
        for k in range(n_chunks):
            b = k & 1
            if k + 1 < n_chunks:
                in_cps[b ^ 1] = pltpu.async_copy(
                    idx_hbm.at[pl.ds(base + (k + 1) * _CHUNK, _CHUNK)],
                    idx_bufs[b ^ 1],
                    sins[b ^ 1],
                )
            in_cps[b].wait()
            if out_cps[b] is not None:
                out_cps[b].wait()
            iv_buf = idx_bufs[b]
            ov_buf = out_bufs[b]

            @plsc.parallel_loop(0, _CHUNK, step=1, unroll=4)
            def body(r, iv_buf=iv_buf, ov_buf=ov_buf):
                for c in col_offs:
                    iv = iv_buf[r, pl.ds(c, _LANES)]
                    ov_buf[r, pl.ds(c, _LANES)] = plsc.load_gather(table_v, [iv])

            out_cps[b] = pltpu.async_copy(
                ov_buf, out_hbm.at[pl.ds(base + k * _CHUNK, _CHUNK)], souts[b]
            )
        for cp in out_cps:
            if cp is not None:
                cp.wait()

    return sc_gather


def kernel(propensities, position):
    n_rows, n_cols = position.shape
    idx = position.astype(jnp.int32)
    table = propensities.astype(jnp.float32)

    info = plsc.get_sparse_core_info()
    num_workers = info.num_cores * info.num_subcores
    fn = _sc_gather_call(n_rows, n_cols, table.shape[0], num_workers, info.num_cores)
    return fn(table, idx)


# speedup vs baseline: 2.5883x; 1.5002x over previous
import functools

import jax
import jax.numpy as jnp
from jax import lax
from jax.experimental import pallas as pl
from jax.experimental.pallas import tpu as pltpu
from jax.experimental.pallas import tpu_sc as plsc

_LANES = 16
_CHUNK = 128


def _sc_gather_call(n_rows, n_cols, n_table, num_workers, num_cores):
    cols_per_w = n_cols // num_workers
    n_chunks = cols_per_w // _CHUNK
    vecs_per_row = _CHUNK // _LANES
    mesh = plsc.VectorSubcoreMesh(core_axis_name="c", subcore_axis_name="s")

    @functools.partial(
        pl.kernel,
        mesh=mesh,
        out_type=jax.ShapeDtypeStruct((n_rows, n_cols), jnp.float32),
        scratch_types=[
            pltpu.VMEM((n_table,), jnp.float32),
            pltpu.VMEM((n_rows, _CHUNK), jnp.int32),
            pltpu.VMEM((n_rows, _CHUNK), jnp.int32),
            pltpu.VMEM((n_rows, _CHUNK), jnp.float32),
            pltpu.VMEM((n_rows, _CHUNK), jnp.float32),
            pltpu.SemaphoreType.DMA,
            pltpu.SemaphoreType.DMA,
            pltpu.SemaphoreType.DMA,
            pltpu.SemaphoreType.DMA,
        ],
        compiler_params=pltpu.CompilerParams(
            needs_layout_passes=False, disable_bounds_checks=True
        ),
    )
    def sc_gather(
        table_hbm, idx_hbm, out_hbm,
        table_v, idx_v0, idx_v1, out_v0, out_v1, sin0, sin1, sout0, sout1,
    ):
        idx_bufs = (idx_v0, idx_v1)
        out_bufs = (out_v0, out_v1)
        sins = (sin0, sin1)
        souts = (sout0, sout1)
        wid = lax.axis_index("s") * num_cores + lax.axis_index("c")
        base = wid * cols_per_w
        pltpu.sync_copy(table_hbm, table_v)

        in_cps = [None, None]
        out_cps = [None, None]
        in_cps[0] = pltpu.async_copy(
            idx_hbm.at[:, pl.ds(base, _CHUNK)], idx_bufs[0], sins[0]
        )
        for k in range(n_chunks):
            b = k & 1
            if k + 1 < n_chunks:
                in_cps[b ^ 1] = pltpu.async_copy(
                    idx_hbm.at[:, pl.ds(base + (k + 1) * _CHUNK, _CHUNK)],
                    idx_bufs[b ^ 1],
                    sins[b ^ 1],
                )
            in_cps[b].wait()
            if out_cps[b] is not None:
                out_cps[b].wait()
            iv_buf = idx_bufs[b]
            ov_buf = out_bufs[b]

            @plsc.parallel_loop(0, n_rows, step=1, unroll=2)
            def body(r, iv_buf=iv_buf, ov_buf=ov_buf):
                for j in range(vecs_per_row):
                    c = j * _LANES
                    iv = iv_buf[r, pl.ds(c, _LANES)]
                    ov_buf[r, pl.ds(c, _LANES)] = plsc.load_gather(table_v, [iv])

            out_cps[b] = pltpu.async_copy(
                ov_buf, out_hbm.at[:, pl.ds(base + k * _CHUNK, _CHUNK)], souts[b]
            )
        for cp in out_cps:
            if cp is not None:
                cp.wait()

    return sc_gather


def kernel(propensities, position):
    n_rows, n_cols = position.shape
    idx_t = position.astype(jnp.int32).T
    table = propensities.astype(jnp.float32)

    info = plsc.get_sparse_core_info()
    num_workers = info.num_cores * info.num_subcores
    fn = _sc_gather_call(n_cols, n_rows, table.shape[0], num_workers, info.num_cores)
    return fn(table, idx_t).T


# speedup vs baseline: 2.6006x; 1.0048x over previous
import functools

import jax
import jax.numpy as jnp
from jax import lax
from jax.experimental import pallas as pl
from jax.experimental.pallas import tpu as pltpu
from jax.experimental.pallas import tpu_sc as plsc

_LANES = 16
_CHUNK = 128


def _sc_gather_call(n_rows, n_cols, n_table, num_workers, num_cores):
    cols_per_w = n_cols // num_workers
    n_chunks = cols_per_w // _CHUNK
    vecs_per_row = _CHUNK // _LANES
    mesh = plsc.VectorSubcoreMesh(core_axis_name="c", subcore_axis_name="s")

    @functools.partial(
        pl.kernel,
        mesh=mesh,
        out_type=jax.ShapeDtypeStruct((n_rows, n_cols), jnp.float32),
        scratch_types=[
            pltpu.VMEM((n_table,), jnp.float32),
            pltpu.VMEM((n_rows, _CHUNK), jnp.int32),
            pltpu.VMEM((n_rows, _CHUNK), jnp.int32),
            pltpu.VMEM((n_rows, _CHUNK), jnp.float32),
            pltpu.VMEM((n_rows, _CHUNK), jnp.float32),
            pltpu.SemaphoreType.DMA,
            pltpu.SemaphoreType.DMA,
            pltpu.SemaphoreType.DMA,
            pltpu.SemaphoreType.DMA,
        ],
        compiler_params=pltpu.CompilerParams(
            needs_layout_passes=False,
            disable_bounds_checks=True,
            skip_device_barrier=True,
        ),
    )
    def sc_gather(
        table_hbm, idx_hbm, out_hbm,
        table_v, idx_v0, idx_v1, out_v0, out_v1, sin0, sin1, sout0, sout1,
    ):
        idx_bufs = (idx_v0, idx_v1)
        out_bufs = (out_v0, out_v1)
        sins = (sin0, sin1)
        souts = (sout0, sout1)
        wid = lax.axis_index("s") * num_cores + lax.axis_index("c")
        base = wid * cols_per_w
        pltpu.sync_copy(table_hbm, table_v)

        in_cps = [None, None]
        out_cps = [None, None]
        in_cps[0] = pltpu.async_copy(
            idx_hbm.at[:, pl.ds(base, _CHUNK)], idx_bufs[0], sins[0]
        )
        for k in range(n_chunks):
            b = k & 1
            if k + 1 < n_chunks:
                in_cps[b ^ 1] = pltpu.async_copy(
                    idx_hbm.at[:, pl.ds(base + (k + 1) * _CHUNK, _CHUNK)],
                    idx_bufs[b ^ 1],
                    sins[b ^ 1],
                )
            in_cps[b].wait()
            if out_cps[b] is not None:
                out_cps[b].wait()
            iv_buf = idx_bufs[b]
            ov_buf = out_bufs[b]

            @plsc.parallel_loop(0, n_rows, step=1, unroll=2)
            def body(r, iv_buf=iv_buf, ov_buf=ov_buf):
                for j in range(vecs_per_row):
                    c = j * _LANES
                    iv = iv_buf[r, pl.ds(c, _LANES)]
                    ov_buf[r, pl.ds(c, _LANES)] = plsc.load_gather(table_v, [iv])

            out_cps[b] = pltpu.async_copy(
                ov_buf, out_hbm.at[:, pl.ds(base + k * _CHUNK, _CHUNK)], souts[b]
            )
        for cp in out_cps:
            if cp is not None:
                cp.wait()

    return sc_gather


def kernel(propensities, position):
    n_rows, n_cols = position.shape
    idx_t = position.astype(jnp.int32).T
    table = propensities.astype(jnp.float32)

    info = plsc.get_sparse_core_info()
    num_workers = info.num_cores * info.num_subcores
    fn = _sc_gather_call(n_cols, n_rows, table.shape[0], num_workers, info.num_cores)
    return fn(table, idx_t).T
